# baseline (speedup 1.0000x reference)
import functools

import jax
import jax.numpy as jnp
from jax import lax
from jax.experimental import pallas as pl
from jax.experimental.pallas import tpu as pltpu
from jax.experimental.pallas import tpu_sc as plsc

_MOMENTUM = 0.75
_NC, _NS, _L = 2, 16, 16
_NW = _NC * _NS

_COLS_A = 1024
_COLS_B = 1024


def _g_body(x_ref, t_ref, g_ref, c_ref):
    g = jnp.abs(x_ref[...] - t_ref[...]).mean(axis=0)
    g_ref[...] = g
    one = jnp.float32(1.0)
    zero = jnp.float32(0.0)
    c0 = jnp.sum(jnp.where(g < 0.1, one, zero))
    c1 = jnp.sum(jnp.where((g >= 0.1) & (g < 0.2), one, zero))
    lane = lax.broadcasted_iota(jnp.int32, (1, 1, 128), 2)
    c_ref[...] = jnp.where(lane == 0, c0, jnp.where(lane == 1, c1, zero))


def _loss_body(w_ref, x_ref, t_ref, o_ref):
    @pl.when(pl.program_id(0) == 0)
    def _():
        o_ref[...] = jnp.zeros_like(o_ref)

    p = x_ref[...]
    t = t_ref[...]
    bce = -(t * jnp.log(p) + (1.0 - t) * jnp.log(1.0 - p))
    o_ref[...] += jnp.sum(bce.mean(axis=0) * w_ref[...])[None, None]


def _make_sc_weights(bs, cla, nparts):
    chunk = bs // _NW
    tot = float(bs * cla)
    mesh = plsc.VectorSubcoreMesh(core_axis_name="c", subcore_axis_name="s")

    @functools.partial(
        pl.kernel,
        mesh=mesh,
        out_type=jax.ShapeDtypeStruct((bs,), jnp.float32),
        scratch_types=[
            pltpu.VMEM((chunk,), jnp.float32),
            pltpu.VMEM((chunk,), jnp.float32),
            pltpu.VMEM((nparts * 128,), jnp.float32),
        ],
    )
    def sc_weights(g_hbm, cnt_hbm, w_hbm, g_v, w_v, c_v):
        wid = lax.axis_index("s") * _NC + lax.axis_index("c")
        lane = lax.broadcasted_iota(jnp.int32, (_L,), 0)
        base = wid * chunk
        pltpu.sync_copy(cnt_hbm, c_v)
        pltpu.sync_copy(g_hbm.at[pl.ds(base, chunk)], g_v)

        total = jnp.zeros((_L,), jnp.float32)
        for i in range(nparts):
            total = total + c_v[pl.ds(i * 128, _L)]

        def allsum(v):
            for sh in (8, 4, 2, 1):
                v = v + v.at[lane ^ sh].get(mode="promise_in_bounds")
            return v

        zero = jnp.float32(0.0)
        c0 = allsum(jnp.where(lane == 0, total, zero))
        c1 = allsum(jnp.where(lane == 1, total, zero))
        w0 = jnp.float32(1.0) * tot / jnp.maximum((1.0 - _MOMENTUM) * c0, 1e-12)
        w1 = jnp.float32(0.75) * tot / jnp.maximum((1.0 - _MOMENTUM) * c1, 1e-12)
        s = c0 * w0 * w0 + c1 * w1 * w1

        def weight_body(i, _):
            gc = g_v[pl.ds(i * _L, _L)]
            sel = jnp.where(gc < 0.1, w0, jnp.where(gc < 0.2, w1, zero))
            wr = sel * sel / s
            wr = jnp.where(wr < 1e-6, zero, wr)
            w_v[pl.ds(i * _L, _L)] = wr
            return 0

        lax.fori_loop(0, chunk // _L, weight_body, 0)
        pltpu.sync_copy(w_v, w_hbm.at[pl.ds(base, chunk)])

    return sc_weights


def kernel(inputs, targets):
    bs, cla = inputs.shape
    nparts = bs // _COLS_A
    xt = inputs.T
    tt = targets.T

    g, counts = pl.pallas_call(
        _g_body,
        grid=(nparts,),
        in_specs=[
            pl.BlockSpec((cla, _COLS_A), lambda i: (0, i)),
            pl.BlockSpec((cla, _COLS_A), lambda i: (0, i)),
        ],
        out_specs=[
            pl.BlockSpec((_COLS_A,), lambda i: (i,)),
            pl.BlockSpec((1, 1, 128), lambda i: (i, 0, 0)),
        ],
        out_shape=[
            jax.ShapeDtypeStruct((bs,), jnp.float32),
            jax.ShapeDtypeStruct((nparts, 1, 128), jnp.float32),
        ],
    )(xt, tt)

    def bce_branch(ops):
        g_, cnt_, x_, t_ = ops
        w_ = _make_sc_weights(bs, cla, nparts)(g_, cnt_.reshape(-1))
        out = pl.pallas_call(
            _loss_body,
            grid=(bs // _COLS_B,),
            in_specs=[
                pl.BlockSpec((_COLS_B,), lambda i: (i,)),
                pl.BlockSpec((cla, _COLS_B), lambda i: (0, i)),
                pl.BlockSpec((cla, _COLS_B), lambda i: (0, i)),
            ],
            out_specs=pl.BlockSpec((1, 1), lambda i: (0, 0)),
            out_shape=jax.ShapeDtypeStruct((1, 1), jnp.float32),
        )(w_, x_, t_)
        return out[0, 0]

    def nan_branch(ops):
        return jnp.float32(jnp.nan)

    pred = jnp.sum(lax.slice(counts, (0, 0, 0), (nparts, 1, 2))) > 0.0
    return lax.cond(pred, lambda o: o[0][0] + o[2][0, 0] + o[3][0, 0],
                    lambda o: jnp.float32(jnp.nan),
                    (g, counts, xt, tt))


# speedup vs baseline: 1.0435x; 1.0435x over previous
import jax
import jax.numpy as jnp
from jax import lax
from jax.experimental import pallas as pl

_MOMENTUM = 0.75

_COLS_A = 1024
_COLS_B = 1024


def _g_body(x_ref, t_ref, g_ref, c_ref):
    g = jnp.abs(x_ref[...] - t_ref[...]).mean(axis=0)
    g_ref[...] = g
    one = jnp.float32(1.0)
    zero = jnp.float32(0.0)
    c0 = jnp.sum(jnp.where(g < 0.1, one, zero))
    c1 = jnp.sum(jnp.where((g >= 0.1) & (g < 0.2), one, zero))
    lane = lax.broadcasted_iota(jnp.int32, (1, 1, 128), 2)
    c_ref[...] = jnp.where(lane == 0, c0, jnp.where(lane == 1, c1, zero))


def _make_loss_body(tot):
    def _loss_body(cnt_ref, g_ref, x_ref, t_ref, o_ref):
        @pl.when(pl.program_id(0) == 0)
        def _():
            o_ref[...] = jnp.zeros_like(o_ref)

        total = jnp.sum(cnt_ref[...], axis=0)
        c0 = total[0, 0]
        c1 = total[0, 1]
        w0 = jnp.float32(1.0) * tot / jnp.maximum((1.0 - _MOMENTUM) * c0, 1e-12)
        w1 = jnp.float32(0.75) * tot / jnp.maximum((1.0 - _MOMENTUM) * c1, 1e-12)
        s = c0 * w0 * w0 + c1 * w1 * w1

        g = g_ref[...]
        zero = jnp.float32(0.0)
        sel = jnp.where(g < 0.1, w0, jnp.where(g < 0.2, w1, zero))
        w = sel * sel / s
        w = jnp.where(w < 1e-6, zero, w)

        p = x_ref[...]
        t = t_ref[...]
        bce = -(t * jnp.log(p) + (1.0 - t) * jnp.log(1.0 - p))
        o_ref[...] += jnp.sum(bce.mean(axis=0) * w)[None, None]

    return _loss_body


def kernel(inputs, targets):
    bs, cla = inputs.shape
    nparts = bs // _COLS_A
    xt = inputs.T
    tt = targets.T

    g, counts = pl.pallas_call(
        _g_body,
        grid=(nparts,),
        in_specs=[
            pl.BlockSpec((cla, _COLS_A), lambda i: (0, i)),
            pl.BlockSpec((cla, _COLS_A), lambda i: (0, i)),
        ],
        out_specs=[
            pl.BlockSpec((_COLS_A,), lambda i: (i,)),
            pl.BlockSpec((1, 1, 128), lambda i: (i, 0, 0)),
        ],
        out_shape=[
            jax.ShapeDtypeStruct((bs,), jnp.float32),
            jax.ShapeDtypeStruct((nparts, 1, 128), jnp.float32),
        ],
    )(xt, tt)

    def bce_branch(ops):
        g_, cnt_, x_, t_ = ops
        out = pl.pallas_call(
            _make_loss_body(float(bs * cla)),
            grid=(bs // _COLS_B,),
            in_specs=[
                pl.BlockSpec((nparts, 1, 128), lambda i: (0, 0, 0)),
                pl.BlockSpec((_COLS_B,), lambda i: (i,)),
                pl.BlockSpec((cla, _COLS_B), lambda i: (0, i)),
                pl.BlockSpec((cla, _COLS_B), lambda i: (0, i)),
            ],
            out_specs=pl.BlockSpec((1, 1), lambda i: (0, 0)),
            out_shape=jax.ShapeDtypeStruct((1, 1), jnp.float32),
        )(cnt_, g_, x_, t_)
        return out[0, 0]

    def nan_branch(ops):
        return jnp.float32(jnp.nan)

    pred = jnp.sum(lax.slice(counts, (0, 0, 0), (nparts, 1, 2))) > 0.0
    return lax.cond(pred, bce_branch, nan_branch, (g, counts, xt, tt))
